# Initial kernel scaffold; baseline (speedup 1.0000x reference)
import jax
import jax.numpy as jnp
from jax.experimental import pallas as pl


def kernel(token_ids, padding_mask, embeds_weight):
    raise NotImplementedError("write your pallas kernel here")


# speedup vs baseline: 1.9599x; 1.9599x over previous
import jax
import jax.numpy as jnp
from jax.experimental import pallas as pl

_RATIO = 16


def _body(mask_ref, w_ref, emb_ref, gmask_ref):
    B, Sg, D = emb_ref.shape
    w0 = w_ref[0, :]
    w1 = w_ref[1, :]
    emb_ref[...] = jnp.broadcast_to(w0[None, None, :], (B, Sg, D))
    emb_ref[:, 0, :] = jnp.broadcast_to(w1[None, :], (B, D))
    gmask_ref[...] = jnp.all(mask_ref[...], axis=2)


def kernel(token_ids, padding_mask, embeds_weight):
    B, Sl = padding_mask.shape
    Sg = Sl // _RATIO
    D = embeds_weight.shape[1]
    mask3 = padding_mask.reshape(B, Sg, _RATIO)
    emb, gmask = pl.pallas_call(
        _body,
        out_shape=(
            jax.ShapeDtypeStruct((B, Sg, D), embeds_weight.dtype),
            jax.ShapeDtypeStruct((B, Sg), jnp.bool_),
        ),
    )(mask3, embeds_weight)
    return (emb, gmask)
